# baseline (speedup 1.0000x reference)
import functools

import jax
import jax.numpy as jnp
from jax import lax
from jax.experimental import pallas as pl
from jax.experimental.pallas import tpu as pltpu
from jax.experimental.pallas import tpu_sc as plsc

N = 10000
E = 320000
NFEAT = 128
NHID = 8
HEADS = 8
NCLASS = 40

NC = 2
NS = 16
NW = NC * NS
EPT = E // NW
C = 200
NCHUNK = EPT // C
NPAD = 10240
RPT = NPAD // NS
ZR = 32
HA = 104
HB = 96

TW1 = 80
AW1 = 80
AW2 = 48

_mesh = plsc.VectorSubcoreMesh(
    core_axis_name="c", subcore_axis_name="s", num_cores=NC, num_subcores=NS)


def _zero_acc(zb, accsh, s, width):
    def zrow(r, _):
        for j in range(width // 16):
            zb[r, pl.ds(16 * j, 16)] = jnp.zeros((16,), jnp.float32)
        return 0
    lax.fori_loop(0, ZR, zrow, 0)

    def zcp(k, _):
        pltpu.sync_copy(zb, accsh.at[pl.ds(s * RPT + k * ZR, ZR)])
        return 0
    lax.fori_loop(0, RPT // ZR, zcp, 0)


def _sc1_body(ei_hbm, tab_hbm, ad_hbm, acc_hbm,
              srcall, dstall, rows0, adb0, rows1, adb1, outba, outbb,
              zb, accsh, g0a, g0b, g1a, g1b, soa, sob):
    c = lax.axis_index("c")
    s = lax.axis_index("s")
    wid = c * NS + s
    ebase = wid * EPT

    pltpu.sync_copy(ei_hbm.at[0, pl.ds(ebase, EPT)], srcall)
    pltpu.sync_copy(ei_hbm.at[1, pl.ds(ebase, EPT)], dstall)
    _zero_acc(zb, accsh, s, AW1)
    plsc.subcore_barrier()

    lane = lax.iota(jnp.int32, 16)
    gsets = ((rows0, adb0, g0a, g0b),
             (rows1, adb1, g1a, g1b))

    def fire(k, t):
        rows, adb, ga, gb = gsets[k]
        pltpu.async_copy(tab_hbm.at[srcall.at[pl.ds(t * C, C)]], rows, ga)
        pltpu.async_copy(ad_hbm.at[dstall.at[pl.ds(t * C, C)]], adb, gb)

    def wait_g(k, t):
        rows, adb, ga, gb = gsets[k]
        pltpu.make_async_copy(tab_hbm.at[srcall.at[pl.ds(t * C, C)]], rows, ga).wait()
        pltpu.make_async_copy(ad_hbm.at[dstall.at[pl.ds(t * C, C)]], adb, gb).wait()

    def wait_oa():
        pltpu.make_async_copy(outba, accsh.at[dstall.at[pl.ds(0, HA)]], soa).wait()

    def wait_ob():
        pltpu.make_async_copy(outbb, accsh.at[dstall.at[pl.ds(0, HB)]], sob).wait()

    def compute(k, t, guard):
        rows, adb, ga, gb = gsets[k]

        def half(hb, off, ln, sem):
            @plsc.parallel_loop(0, ln, 1, unroll=8)
            def edge(i):
                e = i + off
                a = rows[e, pl.ds(64, 16)]
                d = adb[e, :]
                sv = a + d
                lr = jnp.where(sv >= 0.0, sv, 0.2 * sv)
                w = jnp.exp(lr)
                for j in range(4):
                    hj = rows[e, pl.ds(16 * j, 16)]
                    hb[i, pl.ds(16 * j, 16)] = hj * w
                hb[i, pl.ds(64, 16)] = w

            pltpu.async_copy(
                hb, accsh.at[dstall.at[pl.ds(t * C + off, ln)]], sem, add=True)

        if guard:
            @pl.when(t > 0)
            def _():
                wait_oa()
        else:
            wait_oa()
        half(outba, 0, HA, soa)
        if guard:
            @pl.when(t > 0)
            def _():
                wait_ob()
        else:
            wait_ob()
        half(outbb, HA, HB, sob)

    fire(0, 0)

    def pair(t2, _):
        t0 = 2 * t2
        fire(1, t0 + 1)
        wait_g(0, t0)
        compute(0, t0, True)
        fire(0, jnp.where(t0 + 2 < NCHUNK, t0 + 2, 0))
        wait_g(1, t0 + 1)
        compute(1, t0 + 1, False)
        return 0
    lax.fori_loop(0, NCHUNK // 2, pair, 0)

    wait_g(0, 0)
    wait_oa()
    wait_ob()

    plsc.subcore_barrier()
    pltpu.sync_copy(accsh.at[pl.ds(s * RPT, RPT)],
                    acc_hbm.at[c, pl.ds(s * RPT, RPT)])


_sc1 = functools.partial(
    pl.kernel,
    out_type=jax.ShapeDtypeStruct((NC, NPAD, AW1), jnp.float32),
    mesh=_mesh,
    scratch_types=[
        pltpu.VMEM((EPT,), jnp.int32),
        pltpu.VMEM((EPT,), jnp.int32),
        pltpu.VMEM((C, TW1), jnp.float32),
        pltpu.VMEM((C, 16), jnp.float32),
        pltpu.VMEM((C, TW1), jnp.float32),
        pltpu.VMEM((C, 16), jnp.float32),
        pltpu.VMEM((HA, AW1), jnp.float32),
        pltpu.VMEM((HB, AW1), jnp.float32),
        pltpu.VMEM((ZR, AW1), jnp.float32),
        pltpu.VMEM_SHARED((NPAD, AW1), jnp.float32),
        pltpu.SemaphoreType.DMA,
        pltpu.SemaphoreType.DMA,
        pltpu.SemaphoreType.DMA,
        pltpu.SemaphoreType.DMA,
        pltpu.SemaphoreType.DMA,
        pltpu.SemaphoreType.DMA,
    ],
    compiler_params=pltpu.CompilerParams(use_tc_tiling_on_sc=False),
)(_sc1_body)


def _sc2_body(ei_hbm, tab_hbm, att_hbm, acc_hbm,
              srcall, dstall, rows0, adb0, rows1, adb1, outba, outbb,
              zb, accsh, g0a, g0b, g1a, g1b, soa, sob):
    c = lax.axis_index("c")
    s = lax.axis_index("s")
    wid = c * NS + s
    ebase = wid * EPT

    pltpu.sync_copy(ei_hbm.at[0, pl.ds(ebase, EPT)], srcall)
    pltpu.sync_copy(ei_hbm.at[1, pl.ds(ebase, EPT)], dstall)
    _zero_acc(zb, accsh, s, AW2)
    plsc.subcore_barrier()

    lane = lax.iota(jnp.int32, 16)
    gsets = ((rows0, adb0, g0a, g0b),
             (rows1, adb1, g1a, g1b))

    def fire(k, t):
        rows, adb, ga, gb = gsets[k]
        pltpu.async_copy(tab_hbm.at[srcall.at[pl.ds(t * C, C)]], rows, ga)
        pltpu.async_copy(att_hbm.at[dstall.at[pl.ds(t * C, C)]], adb, gb)

    def wait_g(k, t):
        rows, adb, ga, gb = gsets[k]
        pltpu.make_async_copy(tab_hbm.at[srcall.at[pl.ds(t * C, C)]], rows, ga).wait()
        pltpu.make_async_copy(att_hbm.at[dstall.at[pl.ds(t * C, C)]], adb, gb).wait()

    def wait_oa():
        pltpu.make_async_copy(outba, accsh.at[dstall.at[pl.ds(0, HA)]], soa).wait()

    def wait_ob():
        pltpu.make_async_copy(outbb, accsh.at[dstall.at[pl.ds(0, HB)]], sob).wait()

    def compute(k, t, guard):
        rows, adb, ga, gb = gsets[k]

        def half(hb, off, ln, sem):
            @plsc.parallel_loop(0, ln, 1, unroll=8)
            def edge(i):
                e = i + off
                t0v = rows[e, pl.ds(0, 16)]
                t1v = rows[e, pl.ds(16, 16)]
                t2v = rows[e, pl.ds(32, 16)]
                dv = adb[e, :]
                sv = jnp.full((16,), t2v[8], jnp.float32) + dv
                lr = jnp.where(sv >= 0.0, sv, 0.2 * sv)
                w = jnp.exp(lr)
                hb[i, pl.ds(0, 16)] = t0v * w
                hb[i, pl.ds(16, 16)] = t1v * w
                hb[i, pl.ds(32, 16)] = jnp.where(lane < 8, t2v * w, w)

            pltpu.async_copy(
                hb, accsh.at[dstall.at[pl.ds(t * C + off, ln)]], sem, add=True)

        if guard:
            @pl.when(t > 0)
            def _():
                wait_oa()
        else:
            wait_oa()
        half(outba, 0, HA, soa)
        if guard:
            @pl.when(t > 0)
            def _():
                wait_ob()
        else:
            wait_ob()
        half(outbb, HA, HB, sob)

    fire(0, 0)

    def pair(t2, _):
        t0 = 2 * t2
        fire(1, t0 + 1)
        wait_g(0, t0)
        compute(0, t0, True)
        fire(0, jnp.where(t0 + 2 < NCHUNK, t0 + 2, 0))
        wait_g(1, t0 + 1)
        compute(1, t0 + 1, False)
        return 0
    lax.fori_loop(0, NCHUNK // 2, pair, 0)

    wait_g(0, 0)
    wait_oa()
    wait_ob()

    plsc.subcore_barrier()
    pltpu.sync_copy(accsh.at[pl.ds(s * RPT, RPT)],
                    acc_hbm.at[c, pl.ds(s * RPT, RPT)])


_sc2 = functools.partial(
    pl.kernel,
    out_type=jax.ShapeDtypeStruct((NC, NPAD, AW2), jnp.float32),
    mesh=_mesh,
    scratch_types=[
        pltpu.VMEM((EPT,), jnp.int32),
        pltpu.VMEM((EPT,), jnp.int32),
        pltpu.VMEM((C, AW2), jnp.float32),
        pltpu.VMEM((C, 16), jnp.float32),
        pltpu.VMEM((C, AW2), jnp.float32),
        pltpu.VMEM((C, 16), jnp.float32),
        pltpu.VMEM((HA, AW2), jnp.float32),
        pltpu.VMEM((HB, AW2), jnp.float32),
        pltpu.VMEM((ZR, AW2), jnp.float32),
        pltpu.VMEM_SHARED((NPAD, AW2), jnp.float32),
        pltpu.SemaphoreType.DMA,
        pltpu.SemaphoreType.DMA,
        pltpu.SemaphoreType.DMA,
        pltpu.SemaphoreType.DMA,
        pltpu.SemaphoreType.DMA,
        pltpu.SemaphoreType.DMA,
    ],
    compiler_params=pltpu.CompilerParams(use_tc_tiling_on_sc=False),
)(_sc2_body)


_B = 1000
_G = N // _B


def _k1_body(x_ref, w1_ref, m1_ref, a1s_ref, a1d_ref,
             tab_ref, h_ref, as_ref, ad_ref):
    h = jnp.dot(x_ref[:, :], w1_ref[:, :], preferred_element_type=jnp.float32)
    tab_ref[:, :] = jnp.dot(h, m1_ref[:, :], preferred_element_type=jnp.float32)
    h_ref[:, :] = h
    as_ref[:, :] = jnp.dot(h, a1s_ref[:, :], preferred_element_type=jnp.float32)
    ad_ref[:, :] = jnp.dot(h, a1d_ref[:, :], preferred_element_type=jnp.float32)


def _full(shape):
    return pl.BlockSpec(shape, lambda i: (0, 0))


def _rows(width):
    return pl.BlockSpec((_B, width), lambda i: (i, 0))


_k1 = pl.pallas_call(
    _k1_body,
    grid=(_G,),
    in_specs=[_rows(NFEAT), _full((NFEAT, 64)), _full((64, TW1)),
              _full((64, 16)), _full((64, 16))],
    out_specs=[_rows(TW1), _rows(64), _rows(16), _rows(16)],
    out_shape=[
        jax.ShapeDtypeStruct((N, TW1), jnp.float32),
        jax.ShapeDtypeStruct((N, 64), jnp.float32),
        jax.ShapeDtypeStruct((N, 16), jnp.float32),
        jax.ShapeDtypeStruct((N, 16), jnp.float32),
    ],
)


def _k2_body(acc0_ref, acc1_ref, h1_ref, as_ref, ad_ref, sh_ref, eden_ref,
             e16_ref, b1_ref, w2p_ref, a2p_ref, h2_ref, att2_ref):
    accs = acc0_ref[0] + acc1_ref[0]
    num = jnp.dot(accs, sh_ref[:, :], preferred_element_type=jnp.float32)
    den = jnp.dot(accs, eden_ref[:, :], preferred_element_type=jnp.float32)
    a64 = jnp.dot(as_ref[:, :], e16_ref[:, :], preferred_element_type=jnp.float32)
    d64 = jnp.dot(ad_ref[:, :], e16_ref[:, :], preferred_element_type=jnp.float32)
    sv = a64 + d64
    w64 = jnp.exp(jnp.where(sv >= 0.0, sv, 0.2 * sv))
    num = num + h1_ref[:, :] * w64
    den = den + w64
    o = num / den + b1_ref[:, :]
    h1e = jnp.where(o > 0.0, o, jnp.exp(o) - 1.0)
    h2 = jnp.dot(h1e, w2p_ref[:, :], preferred_element_type=jnp.float32)
    h2_ref[:, :] = h2
    att2_ref[:, :] = jnp.dot(h2, a2p_ref[:, :], preferred_element_type=jnp.float32)


def _acc_spec(width, core):
    return pl.BlockSpec((1, _B, width), lambda i, _c=core: (_c, i, 0))


_k2 = pl.pallas_call(
    _k2_body,
    grid=(_G,),
    in_specs=[_acc_spec(AW1, 0), _acc_spec(AW1, 1), _rows(64), _rows(16), _rows(16),
              _full((AW1, 64)), _full((AW1, 64)), _full((16, 64)),
              _full((1, 64)), _full((64, AW2)), _full((AW2, 16))],
    out_specs=[_rows(AW2), _rows(16)],
    out_shape=[
        jax.ShapeDtypeStruct((N, AW2), jnp.float32),
        jax.ShapeDtypeStruct((N, 16), jnp.float32),
    ],
)


def _k3_body(acc0_ref, acc1_ref, h2_ref, att2_ref, c0_ref, dm_ref,
             b2_ref, out_ref):
    accs = acc0_ref[0] + acc1_ref[0]
    asp = jnp.dot(h2_ref[:, :], dm_ref[:, :], preferred_element_type=jnp.float32)
    dsp = jnp.dot(att2_ref[:, :], c0_ref[:, :], preferred_element_type=jnp.float32)
    sv = asp + dsp
    w = jnp.exp(jnp.where(sv >= 0.0, sv, 0.2 * sv))
    mask = lax.broadcasted_iota(jnp.int32, (_B, AW2), 1) < NCLASS
    num = jnp.where(mask, accs, 0.0) + h2_ref[:, :] * w
    den = jnp.dot(accs, dm_ref[:, :], preferred_element_type=jnp.float32) + w
    logits = num / den + b2_ref[:, :]
    lm = jnp.where(mask, logits, -jnp.inf)
    m = jnp.max(lm, axis=1, keepdims=True)
    ex = jnp.exp(lm - m)
    ss = jnp.sum(ex, axis=1, keepdims=True)
    res = lm - m - jnp.log(ss)
    out_ref[:, :] = lax.slice(res, (0, 0), (_B, NCLASS))


_k3 = pl.pallas_call(
    _k3_body,
    grid=(_G,),
    in_specs=[_acc_spec(AW2, 0), _acc_spec(AW2, 1), _rows(AW2), _rows(16),
              _full((16, AW2)), _full((AW2, AW2)), _full((1, AW2))],
    out_specs=_rows(NCLASS),
    out_shape=jax.ShapeDtypeStruct((N, NCLASS), jnp.float32),
)


def kernel(x, edge_index, W1, att_src1, att_dst1, b1, W2, att_src2, att_dst2, b2):
    eye8 = jnp.eye(8, dtype=jnp.float32)
    z64_8 = jnp.zeros((64, 8), jnp.float32)
    a1s = jnp.concatenate(
        [(att_src1[0][:, :, None] * eye8[:, None, :]).reshape(64, 8), z64_8], axis=1)
    a1d = jnp.concatenate(
        [(att_dst1[0][:, :, None] * eye8[:, None, :]).reshape(64, 8), z64_8], axis=1)
    perm = (jnp.arange(64) % 8) * 8 + jnp.arange(64) // 8
    p64t = jnp.zeros((64, 64), jnp.float32).at[jnp.arange(64), perm].set(1.0)
    m1 = jnp.concatenate([p64t, a1s[:, :8], a1s[:, :8]], axis=1)
    a1d = jnp.concatenate([a1d[:, :8], a1d[:, :8]], axis=1)
    rep8 = jnp.kron(eye8, jnp.ones((1, 8), jnp.float32))
    e16 = jnp.concatenate([rep8, jnp.zeros((8, 64), jnp.float32)], axis=0)
    sh = jnp.concatenate([p64t.T, jnp.zeros((16, 64), jnp.float32)], axis=0)
    eden = jnp.concatenate(
        [jnp.zeros((64, 64), jnp.float32), rep8, jnp.zeros((8, 64), jnp.float32)],
        axis=0)
    b1r = b1.reshape(1, 64)
    w2p = jnp.concatenate(
        [W2, (W2 @ att_src2[0, 0])[:, None], jnp.zeros((64, 7), jnp.float32)],
        axis=1)
    a2p = jnp.zeros((AW2, 16), jnp.float32).at[:NCLASS, :].set(
        jnp.tile(att_dst2[0, 0][:, None], (1, 16)))
    c0 = jnp.zeros((16, AW2), jnp.float32).at[0, :].set(1.0)
    dm = jnp.zeros((AW2, AW2), jnp.float32).at[NCLASS, :].set(1.0)
    b2r = jnp.concatenate([b2, jnp.zeros((8,), jnp.float32)]).reshape(1, AW2)

    tab1, h1, asT, adT = _k1(x, W1, m1, a1s, a1d)
    acc1 = _sc1(edge_index, tab1, adT)
    h2p, att2 = _k2(acc1, acc1, h1, asT, adT, sh, eden, e16, b1r, w2p, a2p)
    acc2 = _sc2(edge_index, h2p, att2)
    return _k3(acc2, acc2, h2p, att2, c0, dm, b2r)


# speedup vs baseline: 1.0359x; 1.0359x over previous
import functools

import jax
import jax.numpy as jnp
from jax import lax
from jax.experimental import pallas as pl
from jax.experimental.pallas import tpu as pltpu
from jax.experimental.pallas import tpu_sc as plsc

N = 10000
E = 320000
NFEAT = 128
NHID = 8
HEADS = 8
NCLASS = 40

NC = 2
NS = 16
NW = NC * NS
EPT = E // NW
C = 200
NCHUNK = EPT // C
NPAD = 10240
RPT = NPAD // NS
ZR = 32
HA = 104
HB = 96

TW1 = 80
AW1 = 80
AW2 = 48

_mesh = plsc.VectorSubcoreMesh(
    core_axis_name="c", subcore_axis_name="s", num_cores=NC, num_subcores=NS)


def _zero_acc(zb, accsh, s, width):
    def zrow(r, _):
        for j in range(width // 16):
            zb[r, pl.ds(16 * j, 16)] = jnp.zeros((16,), jnp.float32)
        return 0
    lax.fori_loop(0, ZR, zrow, 0)

    def zcp(k, _):
        pltpu.sync_copy(zb, accsh.at[pl.ds(s * RPT + k * ZR, ZR)])
        return 0
    lax.fori_loop(0, RPT // ZR, zcp, 0)


def _sc1_body(ei_hbm, tab_hbm, ad_hbm, acc_hbm,
              srcall, dstall, rows0, adb0, rows1, adb1, outba, outbb,
              zb, accsh, g0a, g0b, g1a, g1b, soa, sob):
    c = lax.axis_index("c")
    s = lax.axis_index("s")
    wid = c * NS + s
    ebase = wid * EPT

    pltpu.sync_copy(ei_hbm.at[0, pl.ds(ebase, EPT)], srcall)
    pltpu.sync_copy(ei_hbm.at[1, pl.ds(ebase, EPT)], dstall)
    _zero_acc(zb, accsh, s, AW1)
    plsc.subcore_barrier()

    lane = lax.iota(jnp.int32, 16)
    gsets = ((rows0, adb0, g0a, g0b),
             (rows1, adb1, g1a, g1b))

    def fire(k, t):
        rows, adb, ga, gb = gsets[k]
        pltpu.async_copy(tab_hbm.at[srcall.at[pl.ds(t * C, C)]], rows, ga)
        pltpu.async_copy(ad_hbm.at[dstall.at[pl.ds(t * C, C)]], adb, gb)

    def wait_g(k, t):
        rows, adb, ga, gb = gsets[k]
        pltpu.make_async_copy(tab_hbm.at[srcall.at[pl.ds(t * C, C)]], rows, ga).wait()
        pltpu.make_async_copy(ad_hbm.at[dstall.at[pl.ds(t * C, C)]], adb, gb).wait()

    def wait_oa():
        pltpu.make_async_copy(outba, accsh.at[dstall.at[pl.ds(0, HA)]], soa).wait()

    def wait_ob():
        pltpu.make_async_copy(outbb, accsh.at[dstall.at[pl.ds(0, HB)]], sob).wait()

    def compute(k, t, guard):
        rows, adb, ga, gb = gsets[k]

        def half(hb, off, ln, sem):
            @plsc.parallel_loop(0, ln, 1, unroll=8)
            def edge(i):
                e = i + off
                a = rows[e, pl.ds(64, 16)]
                d = adb[e, :]
                sv = a + d
                lr = jnp.where(sv >= 0.0, sv, 0.2 * sv)
                w = jnp.exp(lr)
                for j in range(4):
                    hj = rows[e, pl.ds(16 * j, 16)]
                    hb[i, pl.ds(16 * j, 16)] = hj * w
                hb[i, pl.ds(64, 16)] = w

            pltpu.async_copy(
                hb, accsh.at[dstall.at[pl.ds(t * C + off, ln)]], sem, add=True)

        if guard:
            @pl.when(t > 0)
            def _():
                wait_oa()
        else:
            wait_oa()
        half(outba, 0, HA, soa)
        if guard:
            @pl.when(t > 0)
            def _():
                wait_ob()
        else:
            wait_ob()
        half(outbb, HA, HB, sob)

    fire(0, 0)

    def pair(t2, _):
        t0 = 2 * t2
        fire(1, t0 + 1)
        wait_g(0, t0)
        compute(0, t0, True)
        fire(0, jnp.where(t0 + 2 < NCHUNK, t0 + 2, 0))
        wait_g(1, t0 + 1)
        compute(1, t0 + 1, False)
        return 0
    lax.fori_loop(0, NCHUNK // 2, pair, 0)

    wait_g(0, 0)
    wait_oa()
    wait_ob()

    plsc.subcore_barrier()
    pltpu.sync_copy(accsh.at[pl.ds(s * RPT, RPT)],
                    acc_hbm.at[c, pl.ds(s * RPT, RPT)])


_sc1 = functools.partial(
    pl.kernel,
    out_type=jax.ShapeDtypeStruct((NC, NPAD, AW1), jnp.float32),
    mesh=_mesh,
    scratch_types=[
        pltpu.VMEM((EPT,), jnp.int32),
        pltpu.VMEM((EPT,), jnp.int32),
        pltpu.VMEM((C, TW1), jnp.float32),
        pltpu.VMEM((C, 16), jnp.float32),
        pltpu.VMEM((C, TW1), jnp.float32),
        pltpu.VMEM((C, 16), jnp.float32),
        pltpu.VMEM((HA, AW1), jnp.float32),
        pltpu.VMEM((HB, AW1), jnp.float32),
        pltpu.VMEM((ZR, AW1), jnp.float32),
        pltpu.VMEM_SHARED((NPAD, AW1), jnp.float32),
        pltpu.SemaphoreType.DMA,
        pltpu.SemaphoreType.DMA,
        pltpu.SemaphoreType.DMA,
        pltpu.SemaphoreType.DMA,
        pltpu.SemaphoreType.DMA,
        pltpu.SemaphoreType.DMA,
    ],
    compiler_params=pltpu.CompilerParams(use_tc_tiling_on_sc=False),
)(_sc1_body)


def _sc2_body(ei_hbm, tab_hbm, att_hbm, acc_hbm,
              srcall, dstall, rows0, adb0, rows1, adb1, outba, outbb,
              zb, accsh, g0a, g0b, g1a, g1b, soa, sob):
    c = lax.axis_index("c")
    s = lax.axis_index("s")
    wid = c * NS + s
    ebase = wid * EPT

    pltpu.sync_copy(ei_hbm.at[0, pl.ds(ebase, EPT)], srcall)
    pltpu.sync_copy(ei_hbm.at[1, pl.ds(ebase, EPT)], dstall)
    _zero_acc(zb, accsh, s, AW2)
    plsc.subcore_barrier()

    lane = lax.iota(jnp.int32, 16)
    gsets = ((rows0, adb0, g0a, g0b),
             (rows1, adb1, g1a, g1b))

    def fire(k, t):
        rows, adb, ga, gb = gsets[k]
        pltpu.async_copy(tab_hbm.at[srcall.at[pl.ds(t * C, C)]], rows, ga)
        pltpu.async_copy(att_hbm.at[dstall.at[pl.ds(t * C, C)]], adb, gb)

    def wait_g(k, t):
        rows, adb, ga, gb = gsets[k]
        pltpu.make_async_copy(tab_hbm.at[srcall.at[pl.ds(t * C, C)]], rows, ga).wait()
        pltpu.make_async_copy(att_hbm.at[dstall.at[pl.ds(t * C, C)]], adb, gb).wait()

    def wait_oa():
        pltpu.make_async_copy(outba, accsh.at[dstall.at[pl.ds(0, HA)]], soa).wait()

    def wait_ob():
        pltpu.make_async_copy(outbb, accsh.at[dstall.at[pl.ds(0, HB)]], sob).wait()

    def compute(k, t, guard):
        rows, adb, ga, gb = gsets[k]

        def half(hb, off, ln, sem):
            @plsc.parallel_loop(0, ln, 1, unroll=8)
            def edge(i):
                e = i + off
                t0v = rows[e, pl.ds(0, 16)]
                t1v = rows[e, pl.ds(16, 16)]
                t2v = rows[e, pl.ds(32, 16)]
                dv = adb[e, :]
                sv = jnp.full((16,), t2v[8], jnp.float32) + dv
                lr = jnp.where(sv >= 0.0, sv, 0.2 * sv)
                w = jnp.exp(lr)
                hb[i, pl.ds(0, 16)] = t0v * w
                hb[i, pl.ds(16, 16)] = t1v * w
                hb[i, pl.ds(32, 16)] = jnp.where(lane < 8, t2v * w, w)

            pltpu.async_copy(
                hb, accsh.at[dstall.at[pl.ds(t * C + off, ln)]], sem, add=True)

        if guard:
            @pl.when(t > 0)
            def _():
                wait_oa()
        else:
            wait_oa()
        half(outba, 0, HA, soa)
        if guard:
            @pl.when(t > 0)
            def _():
                wait_ob()
        else:
            wait_ob()
        half(outbb, HA, HB, sob)

    fire(0, 0)

    def pair(t2, _):
        t0 = 2 * t2
        fire(1, t0 + 1)
        wait_g(0, t0)
        compute(0, t0, True)
        fire(0, jnp.where(t0 + 2 < NCHUNK, t0 + 2, 0))
        wait_g(1, t0 + 1)
        compute(1, t0 + 1, False)
        return 0
    lax.fori_loop(0, NCHUNK // 2, pair, 0)

    wait_g(0, 0)
    wait_oa()
    wait_ob()

    plsc.subcore_barrier()
    pltpu.sync_copy(accsh.at[pl.ds(s * RPT, RPT)],
                    acc_hbm.at[c, pl.ds(s * RPT, RPT)])


_sc2 = functools.partial(
    pl.kernel,
    out_type=jax.ShapeDtypeStruct((NC, NPAD, AW2), jnp.float32),
    mesh=_mesh,
    scratch_types=[
        pltpu.VMEM((EPT,), jnp.int32),
        pltpu.VMEM((EPT,), jnp.int32),
        pltpu.VMEM((C, AW2), jnp.float32),
        pltpu.VMEM((C, 16), jnp.float32),
        pltpu.VMEM((C, AW2), jnp.float32),
        pltpu.VMEM((C, 16), jnp.float32),
        pltpu.VMEM((HA, AW2), jnp.float32),
        pltpu.VMEM((HB, AW2), jnp.float32),
        pltpu.VMEM((ZR, AW2), jnp.float32),
        pltpu.VMEM_SHARED((NPAD, AW2), jnp.float32),
        pltpu.SemaphoreType.DMA,
        pltpu.SemaphoreType.DMA,
        pltpu.SemaphoreType.DMA,
        pltpu.SemaphoreType.DMA,
        pltpu.SemaphoreType.DMA,
        pltpu.SemaphoreType.DMA,
    ],
    compiler_params=pltpu.CompilerParams(use_tc_tiling_on_sc=False),
)(_sc2_body)


_B = 2000
_G = N // _B


def _k1_body(x_ref, w1_ref, m1_ref, a1s_ref, a1d_ref,
             tab_ref, h_ref, as_ref, ad_ref):
    h = jnp.dot(x_ref[:, :], w1_ref[:, :], preferred_element_type=jnp.float32)
    tab_ref[:, :] = jnp.dot(h, m1_ref[:, :], preferred_element_type=jnp.float32)
    h_ref[:, :] = h
    as_ref[:, :] = jnp.dot(h, a1s_ref[:, :], preferred_element_type=jnp.float32)
    ad_ref[:, :] = jnp.dot(h, a1d_ref[:, :], preferred_element_type=jnp.float32)


def _full(shape):
    return pl.BlockSpec(shape, lambda i: (0, 0))


def _rows(width):
    return pl.BlockSpec((_B, width), lambda i: (i, 0))


_k1 = pl.pallas_call(
    _k1_body,
    grid=(_G,),
    in_specs=[_rows(NFEAT), _full((NFEAT, 64)), _full((64, TW1)),
              _full((64, 16)), _full((64, 16))],
    out_specs=[_rows(TW1), _rows(64), _rows(16), _rows(16)],
    out_shape=[
        jax.ShapeDtypeStruct((N, TW1), jnp.float32),
        jax.ShapeDtypeStruct((N, 64), jnp.float32),
        jax.ShapeDtypeStruct((N, 16), jnp.float32),
        jax.ShapeDtypeStruct((N, 16), jnp.float32),
    ],
)


def _k2_body(acc0_ref, acc1_ref, h1_ref, as_ref, ad_ref, sh_ref, eden_ref,
             e16_ref, b1_ref, w2p_ref, a2p_ref, h2_ref, att2_ref):
    accs = acc0_ref[0] + acc1_ref[0]
    num = jnp.dot(accs, sh_ref[:, :], preferred_element_type=jnp.float32)
    den = jnp.dot(accs, eden_ref[:, :], preferred_element_type=jnp.float32)
    a64 = jnp.dot(as_ref[:, :], e16_ref[:, :], preferred_element_type=jnp.float32)
    d64 = jnp.dot(ad_ref[:, :], e16_ref[:, :], preferred_element_type=jnp.float32)
    sv = a64 + d64
    w64 = jnp.exp(jnp.where(sv >= 0.0, sv, 0.2 * sv))
    num = num + h1_ref[:, :] * w64
    den = den + w64
    o = num / den + b1_ref[:, :]
    h1e = jnp.where(o > 0.0, o, jnp.exp(o) - 1.0)
    h2 = jnp.dot(h1e, w2p_ref[:, :], preferred_element_type=jnp.float32)
    h2_ref[:, :] = h2
    att2_ref[:, :] = jnp.dot(h2, a2p_ref[:, :], preferred_element_type=jnp.float32)


def _acc_spec(width, core):
    return pl.BlockSpec((1, _B, width), lambda i, _c=core: (_c, i, 0))


_k2 = pl.pallas_call(
    _k2_body,
    grid=(_G,),
    in_specs=[_acc_spec(AW1, 0), _acc_spec(AW1, 1), _rows(64), _rows(16), _rows(16),
              _full((AW1, 64)), _full((AW1, 64)), _full((16, 64)),
              _full((1, 64)), _full((64, AW2)), _full((AW2, 16))],
    out_specs=[_rows(AW2), _rows(16)],
    out_shape=[
        jax.ShapeDtypeStruct((N, AW2), jnp.float32),
        jax.ShapeDtypeStruct((N, 16), jnp.float32),
    ],
)


def _k3_body(acc0_ref, acc1_ref, h2_ref, att2_ref, c0_ref, dm_ref,
             b2_ref, out_ref):
    accs = acc0_ref[0] + acc1_ref[0]
    asp = jnp.dot(h2_ref[:, :], dm_ref[:, :], preferred_element_type=jnp.float32)
    dsp = jnp.dot(att2_ref[:, :], c0_ref[:, :], preferred_element_type=jnp.float32)
    sv = asp + dsp
    w = jnp.exp(jnp.where(sv >= 0.0, sv, 0.2 * sv))
    mask = lax.broadcasted_iota(jnp.int32, (_B, AW2), 1) < NCLASS
    num = jnp.where(mask, accs, 0.0) + h2_ref[:, :] * w
    den = jnp.dot(accs, dm_ref[:, :], preferred_element_type=jnp.float32) + w
    logits = num / den + b2_ref[:, :]
    lm = jnp.where(mask, logits, -jnp.inf)
    m = jnp.max(lm, axis=1, keepdims=True)
    ex = jnp.exp(lm - m)
    ss = jnp.sum(ex, axis=1, keepdims=True)
    res = lm - m - jnp.log(ss)
    out_ref[:, :] = lax.slice(res, (0, 0), (_B, NCLASS))


_k3 = pl.pallas_call(
    _k3_body,
    grid=(_G,),
    in_specs=[_acc_spec(AW2, 0), _acc_spec(AW2, 1), _rows(AW2), _rows(16),
              _full((16, AW2)), _full((AW2, AW2)), _full((1, AW2))],
    out_specs=_rows(NCLASS),
    out_shape=jax.ShapeDtypeStruct((N, NCLASS), jnp.float32),
)


def kernel(x, edge_index, W1, att_src1, att_dst1, b1, W2, att_src2, att_dst2, b2):
    eye8 = jnp.eye(8, dtype=jnp.float32)
    z64_8 = jnp.zeros((64, 8), jnp.float32)
    a1s = jnp.concatenate(
        [(att_src1[0][:, :, None] * eye8[:, None, :]).reshape(64, 8), z64_8], axis=1)
    a1d = jnp.concatenate(
        [(att_dst1[0][:, :, None] * eye8[:, None, :]).reshape(64, 8), z64_8], axis=1)
    perm = (jnp.arange(64) % 8) * 8 + jnp.arange(64) // 8
    p64t = jnp.zeros((64, 64), jnp.float32).at[jnp.arange(64), perm].set(1.0)
    m1 = jnp.concatenate([p64t, a1s[:, :8], a1s[:, :8]], axis=1)
    a1d = jnp.concatenate([a1d[:, :8], a1d[:, :8]], axis=1)
    rep8 = jnp.kron(eye8, jnp.ones((1, 8), jnp.float32))
    e16 = jnp.concatenate([rep8, jnp.zeros((8, 64), jnp.float32)], axis=0)
    sh = jnp.concatenate([p64t.T, jnp.zeros((16, 64), jnp.float32)], axis=0)
    eden = jnp.concatenate(
        [jnp.zeros((64, 64), jnp.float32), rep8, jnp.zeros((8, 64), jnp.float32)],
        axis=0)
    b1r = b1.reshape(1, 64)
    w2p = jnp.concatenate(
        [W2, (W2 @ att_src2[0, 0])[:, None], jnp.zeros((64, 7), jnp.float32)],
        axis=1)
    a2p = jnp.zeros((AW2, 16), jnp.float32).at[:NCLASS, :].set(
        jnp.tile(att_dst2[0, 0][:, None], (1, 16)))
    c0 = jnp.zeros((16, AW2), jnp.float32).at[0, :].set(1.0)
    dm = jnp.zeros((AW2, AW2), jnp.float32).at[NCLASS, :].set(1.0)
    b2r = jnp.concatenate([b2, jnp.zeros((8,), jnp.float32)]).reshape(1, AW2)

    tab1, h1, asT, adT = _k1(x, W1, m1, a1s, a1d)
    acc1 = _sc1(edge_index, tab1, adT)
    h2p, att2 = _k2(acc1, acc1, h1, asT, adT, sh, eden, e16, b1r, w2p, a2p)
    acc2 = _sc2(edge_index, h2p, att2)
    return _k3(acc2, acc2, h2p, att2, c0, dm, b2r)


# speedup vs baseline: 1.0388x; 1.0027x over previous
import functools

import jax
import jax.numpy as jnp
from jax import lax
from jax.experimental import pallas as pl
from jax.experimental.pallas import tpu as pltpu
from jax.experimental.pallas import tpu_sc as plsc

N = 10000
E = 320000
NFEAT = 128
NHID = 8
HEADS = 8
NCLASS = 40

NC = 2
NS = 16
NW = NC * NS
EPT = E // NW
C = 200
NCHUNK = EPT // C
NPAD = 10240
RPT = NPAD // NS
ZR = 32
HA = 104
HB = 96

TW1 = 80
AW1 = 80
AW2 = 48

_mesh = plsc.VectorSubcoreMesh(
    core_axis_name="c", subcore_axis_name="s", num_cores=NC, num_subcores=NS)


def _zero_acc(zb, accsh, s, width):
    def zrow(r, _):
        for j in range(width // 16):
            zb[r, pl.ds(16 * j, 16)] = jnp.zeros((16,), jnp.float32)
        return 0
    lax.fori_loop(0, ZR, zrow, 0)

    def zcp(k, _):
        pltpu.sync_copy(zb, accsh.at[pl.ds(s * RPT + k * ZR, ZR)])
        return 0
    lax.fori_loop(0, RPT // ZR, zcp, 0)


def _sc1_body(ei_hbm, tab_hbm, ad_hbm, acc_hbm,
              srcall, dstall, rows0, adb0, rows1, adb1, outba, outbb,
              zb, accsh, g0a, g0b, g1a, g1b, soa, sob):
    c = lax.axis_index("c")
    s = lax.axis_index("s")
    wid = c * NS + s
    ebase = wid * EPT

    pltpu.sync_copy(ei_hbm.at[0, pl.ds(ebase, EPT)], srcall)
    pltpu.sync_copy(ei_hbm.at[1, pl.ds(ebase, EPT)], dstall)
    _zero_acc(zb, accsh, s, AW1)
    plsc.subcore_barrier()

    lane = lax.iota(jnp.int32, 16)
    gsets = ((rows0, adb0, g0a, g0b),
             (rows1, adb1, g1a, g1b))

    def fire(k, t):
        rows, adb, ga, gb = gsets[k]
        pltpu.async_copy(tab_hbm.at[srcall.at[pl.ds(t * C, C)]], rows, ga)
        pltpu.async_copy(ad_hbm.at[dstall.at[pl.ds(t * C, C)]], adb, gb)

    def wait_g(k, t):
        rows, adb, ga, gb = gsets[k]
        pltpu.make_async_copy(tab_hbm.at[srcall.at[pl.ds(t * C, C)]], rows, ga).wait()
        pltpu.make_async_copy(ad_hbm.at[dstall.at[pl.ds(t * C, C)]], adb, gb).wait()

    def wait_oa():
        pltpu.make_async_copy(outba, accsh.at[dstall.at[pl.ds(0, HA)]], soa).wait()

    def wait_ob():
        pltpu.make_async_copy(outbb, accsh.at[dstall.at[pl.ds(0, HB)]], sob).wait()

    def compute(k, t, guard):
        rows, adb, ga, gb = gsets[k]

        def half(hb, off, ln, sem):
            @plsc.parallel_loop(0, ln, 1, unroll=8)
            def edge(i):
                e = i + off
                a = rows[e, pl.ds(64, 16)]
                d = adb[e, :]
                sv = a + d
                lr = jnp.where(sv >= 0.0, sv, 0.2 * sv)
                w = jnp.exp(lr)
                for j in range(4):
                    hj = rows[e, pl.ds(16 * j, 16)]
                    hb[i, pl.ds(16 * j, 16)] = hj * w
                hb[i, pl.ds(64, 16)] = w

            pltpu.async_copy(
                hb, accsh.at[dstall.at[pl.ds(t * C + off, ln)]], sem, add=True)

        if guard:
            @pl.when(t > 0)
            def _():
                wait_oa()
        else:
            wait_oa()
        half(outba, 0, HA, soa)
        if guard:
            @pl.when(t > 0)
            def _():
                wait_ob()
        else:
            wait_ob()
        half(outbb, HA, HB, sob)

    fire(0, 0)

    def pair(t2, _):
        t0 = 2 * t2
        fire(1, t0 + 1)
        wait_g(0, t0)
        compute(0, t0, True)
        fire(0, jnp.where(t0 + 2 < NCHUNK, t0 + 2, 0))
        wait_g(1, t0 + 1)
        compute(1, t0 + 1, False)
        return 0
    lax.fori_loop(0, NCHUNK // 2, pair, 0)

    wait_g(0, 0)
    wait_oa()
    wait_ob()

    plsc.subcore_barrier()
    pltpu.sync_copy(accsh.at[pl.ds(s * RPT, RPT)],
                    acc_hbm.at[c, pl.ds(s * RPT, RPT)])


_sc1 = functools.partial(
    pl.kernel,
    out_type=jax.ShapeDtypeStruct((NC, NPAD, AW1), jnp.float32),
    mesh=_mesh,
    scratch_types=[
        pltpu.VMEM((EPT,), jnp.int32),
        pltpu.VMEM((EPT,), jnp.int32),
        pltpu.VMEM((C, TW1), jnp.float32),
        pltpu.VMEM((C, 16), jnp.float32),
        pltpu.VMEM((C, TW1), jnp.float32),
        pltpu.VMEM((C, 16), jnp.float32),
        pltpu.VMEM((HA, AW1), jnp.float32),
        pltpu.VMEM((HB, AW1), jnp.float32),
        pltpu.VMEM((ZR, AW1), jnp.float32),
        pltpu.VMEM_SHARED((NPAD, AW1), jnp.float32),
        pltpu.SemaphoreType.DMA,
        pltpu.SemaphoreType.DMA,
        pltpu.SemaphoreType.DMA,
        pltpu.SemaphoreType.DMA,
        pltpu.SemaphoreType.DMA,
        pltpu.SemaphoreType.DMA,
    ],
    compiler_params=pltpu.CompilerParams(use_tc_tiling_on_sc=False),
)(_sc1_body)


def _sc2_body(ei_hbm, tab_hbm, att_hbm, acc_hbm,
              srcall, dstall, rows0, adb0, rows1, adb1, outba, outbb,
              zb, accsh, g0a, g0b, g1a, g1b, soa, sob):
    c = lax.axis_index("c")
    s = lax.axis_index("s")
    wid = c * NS + s
    ebase = wid * EPT

    pltpu.sync_copy(ei_hbm.at[0, pl.ds(ebase, EPT)], srcall)
    pltpu.sync_copy(ei_hbm.at[1, pl.ds(ebase, EPT)], dstall)
    _zero_acc(zb, accsh, s, AW2)
    plsc.subcore_barrier()

    lane = lax.iota(jnp.int32, 16)
    gsets = ((rows0, adb0, g0a, g0b),
             (rows1, adb1, g1a, g1b))

    def fire(k, t):
        rows, adb, ga, gb = gsets[k]
        pltpu.async_copy(tab_hbm.at[srcall.at[pl.ds(t * C, C)]], rows, ga)
        pltpu.async_copy(att_hbm.at[dstall.at[pl.ds(t * C, C)]], adb, gb)

    def wait_g(k, t):
        rows, adb, ga, gb = gsets[k]
        pltpu.make_async_copy(tab_hbm.at[srcall.at[pl.ds(t * C, C)]], rows, ga).wait()
        pltpu.make_async_copy(att_hbm.at[dstall.at[pl.ds(t * C, C)]], adb, gb).wait()

    def wait_oa():
        pltpu.make_async_copy(outba, accsh.at[dstall.at[pl.ds(0, HA)]], soa).wait()

    def wait_ob():
        pltpu.make_async_copy(outbb, accsh.at[dstall.at[pl.ds(0, HB)]], sob).wait()

    def compute(k, t, guard):
        rows, adb, ga, gb = gsets[k]

        def half(hb, off, ln, sem):
            @plsc.parallel_loop(0, ln, 1, unroll=8)
            def edge(i):
                e = i + off
                t0v = rows[e, pl.ds(0, 16)]
                t1v = rows[e, pl.ds(16, 16)]
                t2v = rows[e, pl.ds(32, 16)]
                dv = adb[e, :]
                sv = jnp.full((16,), t2v[8], jnp.float32) + dv
                lr = jnp.where(sv >= 0.0, sv, 0.2 * sv)
                w = jnp.exp(lr)
                hb[i, pl.ds(0, 16)] = t0v * w
                hb[i, pl.ds(16, 16)] = t1v * w
                hb[i, pl.ds(32, 16)] = jnp.where(lane < 8, t2v * w, w)

            pltpu.async_copy(
                hb, accsh.at[dstall.at[pl.ds(t * C + off, ln)]], sem, add=True)

        if guard:
            @pl.when(t > 0)
            def _():
                wait_oa()
        else:
            wait_oa()
        half(outba, 0, HA, soa)
        if guard:
            @pl.when(t > 0)
            def _():
                wait_ob()
        else:
            wait_ob()
        half(outbb, HA, HB, sob)

    fire(0, 0)

    def pair(t2, _):
        t0 = 2 * t2
        fire(1, t0 + 1)
        wait_g(0, t0)
        compute(0, t0, True)
        fire(0, jnp.where(t0 + 2 < NCHUNK, t0 + 2, 0))
        wait_g(1, t0 + 1)
        compute(1, t0 + 1, False)
        return 0
    lax.fori_loop(0, NCHUNK // 2, pair, 0)

    wait_g(0, 0)
    wait_oa()
    wait_ob()

    plsc.subcore_barrier()
    pltpu.sync_copy(accsh.at[pl.ds(s * RPT, RPT)],
                    acc_hbm.at[c, pl.ds(s * RPT, RPT)])


_sc2 = functools.partial(
    pl.kernel,
    out_type=jax.ShapeDtypeStruct((NC, NPAD, AW2), jnp.float32),
    mesh=_mesh,
    scratch_types=[
        pltpu.VMEM((EPT,), jnp.int32),
        pltpu.VMEM((EPT,), jnp.int32),
        pltpu.VMEM((C, AW2), jnp.float32),
        pltpu.VMEM((C, 16), jnp.float32),
        pltpu.VMEM((C, AW2), jnp.float32),
        pltpu.VMEM((C, 16), jnp.float32),
        pltpu.VMEM((HA, AW2), jnp.float32),
        pltpu.VMEM((HB, AW2), jnp.float32),
        pltpu.VMEM((ZR, AW2), jnp.float32),
        pltpu.VMEM_SHARED((NPAD, AW2), jnp.float32),
        pltpu.SemaphoreType.DMA,
        pltpu.SemaphoreType.DMA,
        pltpu.SemaphoreType.DMA,
        pltpu.SemaphoreType.DMA,
        pltpu.SemaphoreType.DMA,
        pltpu.SemaphoreType.DMA,
    ],
    compiler_params=pltpu.CompilerParams(use_tc_tiling_on_sc=False),
)(_sc2_body)


_B = 5000
_G = N // _B


def _k1_body(x_ref, w1_ref, m1_ref, a1s_ref, a1d_ref,
             tab_ref, h_ref, as_ref, ad_ref):
    h = jnp.dot(x_ref[:, :], w1_ref[:, :], preferred_element_type=jnp.float32)
    tab_ref[:, :] = jnp.dot(h, m1_ref[:, :], preferred_element_type=jnp.float32)
    h_ref[:, :] = h
    as_ref[:, :] = jnp.dot(h, a1s_ref[:, :], preferred_element_type=jnp.float32)
    ad_ref[:, :] = jnp.dot(h, a1d_ref[:, :], preferred_element_type=jnp.float32)


def _full(shape):
    return pl.BlockSpec(shape, lambda i: (0, 0))


def _rows(width):
    return pl.BlockSpec((_B, width), lambda i: (i, 0))


_k1 = pl.pallas_call(
    _k1_body,
    grid=(_G,),
    in_specs=[_rows(NFEAT), _full((NFEAT, 64)), _full((64, TW1)),
              _full((64, 16)), _full((64, 16))],
    out_specs=[_rows(TW1), _rows(64), _rows(16), _rows(16)],
    out_shape=[
        jax.ShapeDtypeStruct((N, TW1), jnp.float32),
        jax.ShapeDtypeStruct((N, 64), jnp.float32),
        jax.ShapeDtypeStruct((N, 16), jnp.float32),
        jax.ShapeDtypeStruct((N, 16), jnp.float32),
    ],
)


def _k2_body(acc0_ref, acc1_ref, h1_ref, as_ref, ad_ref, sh_ref, eden_ref,
             e16_ref, b1_ref, w2p_ref, a2p_ref, h2_ref, att2_ref):
    accs = acc0_ref[0] + acc1_ref[0]
    num = jnp.dot(accs, sh_ref[:, :], preferred_element_type=jnp.float32)
    den = jnp.dot(accs, eden_ref[:, :], preferred_element_type=jnp.float32)
    a64 = jnp.dot(as_ref[:, :], e16_ref[:, :], preferred_element_type=jnp.float32)
    d64 = jnp.dot(ad_ref[:, :], e16_ref[:, :], preferred_element_type=jnp.float32)
    sv = a64 + d64
    w64 = jnp.exp(jnp.where(sv >= 0.0, sv, 0.2 * sv))
    num = num + h1_ref[:, :] * w64
    den = den + w64
    o = num / den + b1_ref[:, :]
    h1e = jnp.where(o > 0.0, o, jnp.exp(o) - 1.0)
    h2 = jnp.dot(h1e, w2p_ref[:, :], preferred_element_type=jnp.float32)
    h2_ref[:, :] = h2
    att2_ref[:, :] = jnp.dot(h2, a2p_ref[:, :], preferred_element_type=jnp.float32)


def _acc_spec(width, core):
    return pl.BlockSpec((1, _B, width), lambda i, _c=core: (_c, i, 0))


_k2 = pl.pallas_call(
    _k2_body,
    grid=(_G,),
    in_specs=[_acc_spec(AW1, 0), _acc_spec(AW1, 1), _rows(64), _rows(16), _rows(16),
              _full((AW1, 64)), _full((AW1, 64)), _full((16, 64)),
              _full((1, 64)), _full((64, AW2)), _full((AW2, 16))],
    out_specs=[_rows(AW2), _rows(16)],
    out_shape=[
        jax.ShapeDtypeStruct((N, AW2), jnp.float32),
        jax.ShapeDtypeStruct((N, 16), jnp.float32),
    ],
)


def _k3_body(acc0_ref, acc1_ref, h2_ref, att2_ref, c0_ref, dm_ref,
             b2_ref, out_ref):
    accs = acc0_ref[0] + acc1_ref[0]
    asp = jnp.dot(h2_ref[:, :], dm_ref[:, :], preferred_element_type=jnp.float32)
    dsp = jnp.dot(att2_ref[:, :], c0_ref[:, :], preferred_element_type=jnp.float32)
    sv = asp + dsp
    w = jnp.exp(jnp.where(sv >= 0.0, sv, 0.2 * sv))
    mask = lax.broadcasted_iota(jnp.int32, (_B, AW2), 1) < NCLASS
    num = jnp.where(mask, accs, 0.0) + h2_ref[:, :] * w
    den = jnp.dot(accs, dm_ref[:, :], preferred_element_type=jnp.float32) + w
    logits = num / den + b2_ref[:, :]
    lm = jnp.where(mask, logits, -jnp.inf)
    m = jnp.max(lm, axis=1, keepdims=True)
    ex = jnp.exp(lm - m)
    ss = jnp.sum(ex, axis=1, keepdims=True)
    res = lm - m - jnp.log(ss)
    out_ref[:, :] = lax.slice(res, (0, 0), (_B, NCLASS))


_k3 = pl.pallas_call(
    _k3_body,
    grid=(_G,),
    in_specs=[_acc_spec(AW2, 0), _acc_spec(AW2, 1), _rows(AW2), _rows(16),
              _full((16, AW2)), _full((AW2, AW2)), _full((1, AW2))],
    out_specs=_rows(NCLASS),
    out_shape=jax.ShapeDtypeStruct((N, NCLASS), jnp.float32),
)


def kernel(x, edge_index, W1, att_src1, att_dst1, b1, W2, att_src2, att_dst2, b2):
    eye8 = jnp.eye(8, dtype=jnp.float32)
    z64_8 = jnp.zeros((64, 8), jnp.float32)
    a1s = jnp.concatenate(
        [(att_src1[0][:, :, None] * eye8[:, None, :]).reshape(64, 8), z64_8], axis=1)
    a1d = jnp.concatenate(
        [(att_dst1[0][:, :, None] * eye8[:, None, :]).reshape(64, 8), z64_8], axis=1)
    perm = (jnp.arange(64) % 8) * 8 + jnp.arange(64) // 8
    p64t = jnp.zeros((64, 64), jnp.float32).at[jnp.arange(64), perm].set(1.0)
    m1 = jnp.concatenate([p64t, a1s[:, :8], a1s[:, :8]], axis=1)
    a1d = jnp.concatenate([a1d[:, :8], a1d[:, :8]], axis=1)
    rep8 = jnp.kron(eye8, jnp.ones((1, 8), jnp.float32))
    e16 = jnp.concatenate([rep8, jnp.zeros((8, 64), jnp.float32)], axis=0)
    sh = jnp.concatenate([p64t.T, jnp.zeros((16, 64), jnp.float32)], axis=0)
    eden = jnp.concatenate(
        [jnp.zeros((64, 64), jnp.float32), rep8, jnp.zeros((8, 64), jnp.float32)],
        axis=0)
    b1r = b1.reshape(1, 64)
    w2p = jnp.concatenate(
        [W2, (W2 @ att_src2[0, 0])[:, None], jnp.zeros((64, 7), jnp.float32)],
        axis=1)
    a2p = jnp.zeros((AW2, 16), jnp.float32).at[:NCLASS, :].set(
        jnp.tile(att_dst2[0, 0][:, None], (1, 16)))
    c0 = jnp.zeros((16, AW2), jnp.float32).at[0, :].set(1.0)
    dm = jnp.zeros((AW2, AW2), jnp.float32).at[NCLASS, :].set(1.0)
    b2r = jnp.concatenate([b2, jnp.zeros((8,), jnp.float32)]).reshape(1, AW2)

    tab1, h1, asT, adT = _k1(x, W1, m1, a1s, a1d)
    acc1 = _sc1(edge_index, tab1, adT)
    h2p, att2 = _k2(acc1, acc1, h1, asT, adT, sh, eden, e16, b1r, w2p, a2p)
    acc2 = _sc2(edge_index, h2p, att2)
    return _k3(acc2, acc2, h2p, att2, c0, dm, b2r)
